# Initial kernel scaffold; baseline (speedup 1.0000x reference)
import functools

import jax
import jax.numpy as jnp
from jax import lax
from jax.experimental import pallas as pl
from jax.experimental.pallas import tpu as pltpu
from jax.experimental.pallas import tpu_sc as plsc

_N = 10000
_D = 128
_H = 128
_C = 64

_NC, _NS = 2, 16
_NW = _NC * _NS
_K = 128
_NP = 10240
_ZR = _NP // _NS
_DEGW = 8
_BN = 1280
_GRID = _NP // _BN

_mesh = plsc.VectorSubcoreMesh(core_axis_name="c", subcore_axis_name="s")


def _deg_body(nchunks, dst_hbm, ones_hbm, zeros_hbm, out_hbm, idx_v, ones_v, shared):
    c = lax.axis_index("c")
    s = lax.axis_index("s")
    wid = c * _NS + s
    epw = nchunks * _K
    pltpu.sync_copy(zeros_hbm.at[pl.ds(s * _ZR, _ZR)], shared.at[pl.ds(s * _ZR, _ZR)])
    pltpu.sync_copy(ones_hbm, ones_v)
    plsc.subcore_barrier()

    def body(i, carry):
        base = wid * epw + i * _K
        pltpu.sync_copy(dst_hbm.at[pl.ds(base, _K)], idx_v)
        pltpu.sync_copy(ones_v, shared.at[idx_v], add=True)
        return carry

    lax.fori_loop(0, nchunks, body, 0)
    plsc.subcore_barrier()
    pltpu.sync_copy(shared.at[pl.ds(s * _ZR, _ZR)], out_hbm.at[c, pl.ds(s * _ZR, _ZR)])


def _agg_body(nchunks, y_hbm, src_hbm, dst_hbm, zeros_hbm, out_hbm,
              idx_s, idx_d, rows, shared, sem):
    c = lax.axis_index("c")
    s = lax.axis_index("s")
    wid = c * _NS + s
    epw = nchunks * _K
    pltpu.sync_copy(zeros_hbm.at[pl.ds(s * _ZR, _ZR)], shared.at[pl.ds(s * _ZR, _ZR)])
    plsc.subcore_barrier()

    def body(i, carry):
        base = wid * epw + i * _K
        pltpu.sync_copy(src_hbm.at[pl.ds(base, _K)], idx_s)
        pltpu.sync_copy(dst_hbm.at[pl.ds(base, _K)], idx_d)
        pltpu.async_copy(y_hbm.at[idx_s], rows, sem).wait()
        pltpu.sync_copy(rows, shared.at[idx_d], add=True)
        return carry

    lax.fori_loop(0, nchunks, body, 0)
    plsc.subcore_barrier()
    pltpu.sync_copy(shared.at[pl.ds(s * _ZR, _ZR)], out_hbm.at[c, pl.ds(s * _ZR, _ZR)])


def _make_deg(nchunks):
    return pl.kernel(
        functools.partial(_deg_body, nchunks),
        out_type=jax.ShapeDtypeStruct((_NC, _NP, _DEGW), jnp.float32),
        mesh=_mesh,
        scratch_types=[
            pltpu.VMEM((_K,), jnp.int32),
            pltpu.VMEM((_K, _DEGW), jnp.float32),
            pltpu.VMEM_SHARED((_NP, _DEGW), jnp.float32),
        ],
    )


def _make_agg(nchunks, width):
    return pl.kernel(
        functools.partial(_agg_body, nchunks),
        out_type=jax.ShapeDtypeStruct((_NC, _NP, width), jnp.float32),
        mesh=_mesh,
        scratch_types=[
            pltpu.VMEM((_K,), jnp.int32),
            pltpu.VMEM((_K,), jnp.int32),
            pltpu.VMEM((_K, width), jnp.float32),
            pltpu.VMEM_SHARED((_NP, width), jnp.float32),
            pltpu.SemaphoreType.DMA,
        ],
    )


def _scale_matmul_body(degp_ref, x_ref, w_ref, y_ref, dis_ref):
    deg = degp_ref[0] + degp_ref[1]
    dis = jnp.where(deg > 0, lax.rsqrt(deg), 0.0)
    dis_ref[...] = dis
    xw = jnp.dot(x_ref[...], w_ref[...], preferred_element_type=jnp.float32)
    y_ref[...] = xw * dis[:, 0:1]


def _mid_body(zp_ref, dis_ref, b1_ref, w2_ref, y2_ref):
    z = zp_ref[0] + zp_ref[1]
    dis = dis_ref[...][:, 0:1]
    h = jnp.maximum(dis * z + b1_ref[...], 0.0)
    y2_ref[...] = jnp.dot(h, w2_ref[...], preferred_element_type=jnp.float32) * dis


def _out_body(zp_ref, dis_ref, b2_ref, o_ref):
    z = zp_ref[0] + zp_ref[1]
    o = dis_ref[...][:, 0:1] * z + b2_ref[...]
    m = jnp.max(o, axis=1, keepdims=True)
    e = jnp.exp(o - m)
    o_ref[...] = o - m - jnp.log(jnp.sum(e, axis=1, keepdims=True))


_scale_matmul = pl.pallas_call(
    _scale_matmul_body,
    grid=(_GRID,),
    in_specs=[
        pl.BlockSpec((_NC, _BN, _DEGW), lambda i: (0, i, 0)),
        pl.BlockSpec((_BN, _D), lambda i: (i, 0)),
        pl.BlockSpec((_D, _H), lambda i: (0, 0)),
    ],
    out_specs=[
        pl.BlockSpec((_BN, _H), lambda i: (i, 0)),
        pl.BlockSpec((_BN, _DEGW), lambda i: (i, 0)),
    ],
    out_shape=[
        jax.ShapeDtypeStruct((_NP, _H), jnp.float32),
        jax.ShapeDtypeStruct((_NP, _DEGW), jnp.float32),
    ],
)

_mid = pl.pallas_call(
    _mid_body,
    grid=(_GRID,),
    in_specs=[
        pl.BlockSpec((_NC, _BN, _H), lambda i: (0, i, 0)),
        pl.BlockSpec((_BN, _DEGW), lambda i: (i, 0)),
        pl.BlockSpec((1, _H), lambda i: (0, 0)),
        pl.BlockSpec((_H, _C), lambda i: (0, 0)),
    ],
    out_specs=pl.BlockSpec((_BN, _C), lambda i: (i, 0)),
    out_shape=jax.ShapeDtypeStruct((_NP, _C), jnp.float32),
)

_outk = pl.pallas_call(
    _out_body,
    grid=(_GRID,),
    in_specs=[
        pl.BlockSpec((_NC, _BN, _C), lambda i: (0, i, 0)),
        pl.BlockSpec((_BN, _DEGW), lambda i: (i, 0)),
        pl.BlockSpec((1, _C), lambda i: (0, 0)),
    ],
    out_specs=pl.BlockSpec((_BN, _C), lambda i: (i, 0)),
    out_shape=jax.ShapeDtypeStruct((_NP, _C), jnp.float32),
)


def kernel(x, edge_index, W1, b1, W2, b2):
    E = edge_index.shape[1]
    etot = E + _N
    epad = -(-etot // (_NW * _K)) * (_NW * _K)
    nchunks = epad // (_NW * _K)

    loop = jnp.arange(_N, dtype=jnp.int32)
    padv = _N + (jnp.arange(epad - etot, dtype=jnp.int32) % 16)
    src_p = jnp.concatenate([edge_index[0].astype(jnp.int32), loop, padv])
    dst_p = jnp.concatenate([edge_index[1].astype(jnp.int32), loop, padv])

    x_p = jnp.pad(x, ((0, _NP - _N), (0, 0)))
    ones8 = jnp.ones((_K, _DEGW), jnp.float32)
    z8 = jnp.zeros((_NP, _DEGW), jnp.float32)
    z128 = jnp.zeros((_NP, _H), jnp.float32)
    z64 = jnp.zeros((_NP, _C), jnp.float32)

    degp = _make_deg(nchunks)(dst_p, ones8, z8)
    y1, dis8 = _scale_matmul(degp, x_p, W1)
    zp1 = _make_agg(nchunks, _H)(y1, src_p, dst_p, z128)
    y2 = _mid(zp1, dis8, b1.reshape(1, _H), W2)
    zp2 = _make_agg(nchunks, _C)(y2, src_p, dst_p, z64)
    out = _outk(zp2, dis8, b2.reshape(1, _C))
    return out[:_N]


# speedup vs baseline: 16.7316x; 16.7316x over previous
import functools

import jax
import jax.numpy as jnp
from jax import lax
from jax.experimental import pallas as pl
from jax.experimental.pallas import tpu as pltpu
from jax.experimental.pallas import tpu_sc as plsc

_N = 10000
_D = 128
_H = 128
_C = 64

_NC, _NS = 2, 16
_NW = _NC * _NS
_K = 128
_NP = 10240
_ZR = _NP // _NS
_DEGW = 16
_BN = 1280
_GRID = _NP // _BN

_mesh = plsc.VectorSubcoreMesh(core_axis_name="c", subcore_axis_name="s")


def _deg_body(nchunks, dst_hbm, out_hbm, idx_buf, hist):
    c = lax.axis_index("c")
    s = lax.axis_index("s")
    wid = c * _NS + s
    epw = nchunks * _K

    def zero(i, carry):
        hist[pl.ds(i * 16, 16)] = jnp.zeros((16,), jnp.float32)
        return carry

    lax.fori_loop(0, _NP // 16, zero, 0)
    pltpu.sync_copy(dst_hbm.at[pl.ds(wid * epw, epw)], idx_buf)

    def body(i, carry):
        idx = idx_buf[pl.ds(i * 16, 16)]
        cnt, last = plsc.scan_count(idx)
        plsc.addupdate_scatter(hist, [idx], cnt.astype(jnp.float32), mask=last)
        return carry

    lax.fori_loop(0, epw // 16, body, 0)
    pltpu.sync_copy(hist, out_hbm.at[wid])


def _agg_body(nchunks, y_hbm, src_hbm, dst_hbm, zeros_hbm, out_hbm,
              idx_s, idx_d, rows, shared, sem):
    c = lax.axis_index("c")
    s = lax.axis_index("s")
    wid = c * _NS + s
    epw = nchunks * _K
    pltpu.sync_copy(zeros_hbm.at[pl.ds(s * _ZR, _ZR)], shared.at[pl.ds(s * _ZR, _ZR)])
    plsc.subcore_barrier()

    def body(i, carry):
        base = wid * epw + i * _K
        pltpu.sync_copy(src_hbm.at[pl.ds(base, _K)], idx_s)
        pltpu.sync_copy(dst_hbm.at[pl.ds(base, _K)], idx_d)
        pltpu.async_copy(y_hbm.at[idx_s], rows, sem).wait()
        pltpu.sync_copy(rows, shared.at[idx_d], add=True)
        return carry

    lax.fori_loop(0, nchunks, body, 0)
    plsc.subcore_barrier()
    pltpu.sync_copy(shared.at[pl.ds(s * _ZR, _ZR)], out_hbm.at[c, pl.ds(s * _ZR, _ZR)])


def _make_deg(nchunks):
    return pl.kernel(
        functools.partial(_deg_body, nchunks),
        out_type=jax.ShapeDtypeStruct((_NW, _NP), jnp.float32),
        mesh=_mesh,
        scratch_types=[
            pltpu.VMEM((nchunks * _K,), jnp.int32),
            pltpu.VMEM((_NP,), jnp.float32),
        ],
        compiler_params=pltpu.CompilerParams(needs_layout_passes=False),
    )


def _make_agg(nchunks, width):
    return pl.kernel(
        functools.partial(_agg_body, nchunks),
        out_type=jax.ShapeDtypeStruct((_NC, _NP, width), jnp.float32),
        mesh=_mesh,
        scratch_types=[
            pltpu.VMEM((_K,), jnp.int32),
            pltpu.VMEM((_K,), jnp.int32),
            pltpu.VMEM((_K, width), jnp.float32),
            pltpu.VMEM_SHARED((_NP, width), jnp.float32),
            pltpu.SemaphoreType.DMA,
        ],
    )


def _scale_matmul_body(degp_ref, x_ref, w_ref, y_ref, dis_ref):
    deg = jnp.sum(degp_ref[...], axis=0)
    dis = jnp.where(deg > 0, lax.rsqrt(deg), 0.0)
    dis_ref[...] = dis[None, :]
    xw = jnp.dot(x_ref[...], w_ref[...], preferred_element_type=jnp.float32)
    y_ref[...] = xw * dis[:, None]


def _mid_body(zp_ref, dis_ref, b1_ref, w2_ref, y2_ref):
    z = zp_ref[0] + zp_ref[1]
    dis = dis_ref[0][:, None]
    h = jnp.maximum(dis * z + b1_ref[...], 0.0)
    y2_ref[...] = jnp.dot(h, w2_ref[...], preferred_element_type=jnp.float32) * dis


def _out_body(zp_ref, dis_ref, b2_ref, o_ref):
    z = (zp_ref[0] + zp_ref[1])[:, : _C]
    dis = dis_ref[0][:, None]
    o = dis * z + b2_ref[...]
    m = jnp.max(o, axis=1, keepdims=True)
    e = jnp.exp(o - m)
    o_ref[...] = o - m - jnp.log(jnp.sum(e, axis=1, keepdims=True))


_scale_matmul = pl.pallas_call(
    _scale_matmul_body,
    grid=(_GRID,),
    in_specs=[
        pl.BlockSpec((_NW, _BN), lambda i: (0, i)),
        pl.BlockSpec((_BN, _D), lambda i: (i, 0)),
        pl.BlockSpec((_D, _H), lambda i: (0, 0)),
    ],
    out_specs=[
        pl.BlockSpec((_BN, _H), lambda i: (i, 0)),
        pl.BlockSpec((1, _BN), lambda i: (0, i)),
    ],
    out_shape=[
        jax.ShapeDtypeStruct((_NP, _H), jnp.float32),
        jax.ShapeDtypeStruct((1, _NP), jnp.float32),
    ],
)

_mid = pl.pallas_call(
    _mid_body,
    grid=(_GRID,),
    in_specs=[
        pl.BlockSpec((_NC, _BN, _H), lambda i: (0, i, 0)),
        pl.BlockSpec((1, _BN), lambda i: (0, i)),
        pl.BlockSpec((1, _H), lambda i: (0, 0)),
        pl.BlockSpec((_H, _H), lambda i: (0, 0)),
    ],
    out_specs=pl.BlockSpec((_BN, _H), lambda i: (i, 0)),
    out_shape=jax.ShapeDtypeStruct((_NP, _H), jnp.float32),
)

_outk = pl.pallas_call(
    _out_body,
    grid=(_GRID,),
    in_specs=[
        pl.BlockSpec((_NC, _BN, _H), lambda i: (0, i, 0)),
        pl.BlockSpec((1, _BN), lambda i: (0, i)),
        pl.BlockSpec((1, _C), lambda i: (0, 0)),
    ],
    out_specs=pl.BlockSpec((_BN, _C), lambda i: (i, 0)),
    out_shape=jax.ShapeDtypeStruct((_NP, _C), jnp.float32),
)


def kernel(x, edge_index, W1, b1, W2, b2):
    E = edge_index.shape[1]
    etot = E + _N
    epad = -(-etot // (_NW * _K)) * (_NW * _K)
    nchunks = epad // (_NW * _K)

    loop = jnp.arange(_N, dtype=jnp.int32)
    padv = _N + (jnp.arange(epad - etot, dtype=jnp.int32) % 16)
    src_p = jnp.concatenate([edge_index[0].astype(jnp.int32), loop, padv])
    dst_p = jnp.concatenate([edge_index[1].astype(jnp.int32), loop, padv])

    x_p = jnp.pad(x, ((0, _NP - _N), (0, 0)))
    z128 = jnp.zeros((_NP, _H), jnp.float32)
    W2p = jnp.pad(W2, ((0, 0), (0, _H - _C)))

    degp = _make_deg(nchunks)(dst_p)
    y1, dis8 = _scale_matmul(degp, x_p, W1)
    zp1 = _make_agg(nchunks, _H)(y1, src_p, dst_p, z128)
    y2 = _mid(zp1, dis8, b1.reshape(1, _H), W2p)
    zp2 = _make_agg(nchunks, _H)(y2, src_p, dst_p, z128)
    out = _outk(zp2, dis8, b2.reshape(1, _C))
    return out[:_N]


# speedup vs baseline: 31.8163x; 1.9016x over previous
import functools

import jax
import jax.numpy as jnp
from jax import lax
from jax.experimental import pallas as pl
from jax.experimental.pallas import tpu as pltpu
from jax.experimental.pallas import tpu_sc as plsc

_N = 10000
_D = 128
_H = 128
_C = 64

_NC, _NS = 2, 16
_NW = _NC * _NS
_K = 128
_NP = 10240
_ZR = _NP // _NS
_DEGW = 16
_BN = 1280
_GRID = _NP // _BN

_mesh = plsc.VectorSubcoreMesh(core_axis_name="c", subcore_axis_name="s")


def _deg_body(nchunks, dst_hbm, out_hbm, idx_buf, hist):
    c = lax.axis_index("c")
    s = lax.axis_index("s")
    wid = c * _NS + s
    epw = nchunks * _K

    def zero(i, carry):
        hist[pl.ds(i * 16, 16)] = jnp.zeros((16,), jnp.float32)
        return carry

    lax.fori_loop(0, _NP // 16, zero, 0)
    pltpu.sync_copy(dst_hbm.at[pl.ds(wid * epw, epw)], idx_buf)

    def body(i, carry):
        idx = idx_buf[pl.ds(i * 16, 16)]
        cnt, last = plsc.scan_count(idx)
        plsc.addupdate_scatter(hist, [idx], cnt.astype(jnp.float32), mask=last)
        return carry

    lax.fori_loop(0, epw // 16, body, 0)
    pltpu.sync_copy(hist, out_hbm.at[wid])


_NBUF = 2


def _agg_body(nchunks, y_hbm, srcm_hbm, dst_hbm, zeros_hbm, out_hbm,
              sidx, didx0, didx1, rows, shared, gsem0, gsem1, dsem0, dsem1):
    c = lax.axis_index("c")
    s = lax.axis_index("s")
    wid = c * _NS + s
    epw = nchunks * _K
    didxs = [didx0, didx1]
    gsems = [gsem0, gsem1]
    dsems = [dsem0, dsem1]
    ngrp = nchunks // _NBUF

    pltpu.sync_copy(zeros_hbm.at[pl.ds(s * _ZR, _ZR)], shared.at[pl.ds(s * _ZR, _ZR)])
    pltpu.sync_copy(srcm_hbm.at[wid], sidx)
    plsc.subcore_barrier()

    for b in range(_NBUF):
        pltpu.async_copy(dst_hbm.at[pl.ds(wid * epw + b * _K, _K)], didxs[b], dsems[b])
        pltpu.async_copy(y_hbm.at[sidx.at[b]], rows.at[b], gsems[b])

    def outer(g, carry):
        for b in range(_NBUF):
            j = g * _NBUF + b
            pltpu.make_async_copy(
                dst_hbm.at[pl.ds(wid * epw + j * _K, _K)], didxs[b], dsems[b]
            ).wait()
            pltpu.make_async_copy(y_hbm.at[sidx.at[j]], rows.at[b], gsems[b]).wait()
            pltpu.sync_copy(rows.at[b], shared.at[didxs[b]], add=True)

            @pl.when(g < ngrp - 1)
            def _():
                pltpu.async_copy(
                    dst_hbm.at[pl.ds(wid * epw + (j + _NBUF) * _K, _K)],
                    didxs[b], dsems[b],
                )
                pltpu.async_copy(y_hbm.at[sidx.at[j + _NBUF]], rows.at[b], gsems[b])

        return carry

    lax.fori_loop(0, ngrp, outer, 0)
    plsc.subcore_barrier()
    pltpu.sync_copy(shared.at[pl.ds(s * _ZR, _ZR)], out_hbm.at[c, pl.ds(s * _ZR, _ZR)])


def _make_deg(nchunks):
    return pl.kernel(
        functools.partial(_deg_body, nchunks),
        out_type=jax.ShapeDtypeStruct((_NW, _NP), jnp.float32),
        mesh=_mesh,
        scratch_types=[
            pltpu.VMEM((nchunks * _K,), jnp.int32),
            pltpu.VMEM((_NP,), jnp.float32),
        ],
        compiler_params=pltpu.CompilerParams(needs_layout_passes=False),
    )


def _make_agg(nchunks, width):
    return pl.kernel(
        functools.partial(_agg_body, nchunks),
        out_type=jax.ShapeDtypeStruct((_NC, _NP, width), jnp.float32),
        mesh=_mesh,
        scratch_types=[
            pltpu.VMEM((nchunks, _K), jnp.int32),
            pltpu.VMEM((_K,), jnp.int32),
            pltpu.VMEM((_K,), jnp.int32),
            pltpu.VMEM((_NBUF, _K, width), jnp.float32),
            pltpu.VMEM_SHARED((_NP, width), jnp.float32),
            pltpu.SemaphoreType.DMA,
            pltpu.SemaphoreType.DMA,
            pltpu.SemaphoreType.DMA,
            pltpu.SemaphoreType.DMA,
        ],
    )


def _scale_matmul_body(degp_ref, x_ref, w_ref, y_ref, dis_ref):
    deg = jnp.sum(degp_ref[...], axis=0)
    dis = jnp.where(deg > 0, lax.rsqrt(deg), 0.0)
    dis_ref[...] = dis[None, :]
    xw = jnp.dot(x_ref[...], w_ref[...], preferred_element_type=jnp.float32)
    y_ref[...] = xw * dis[:, None]


def _mid_body(zp_ref, dis_ref, b1_ref, w2_ref, y2_ref):
    z = zp_ref[0] + zp_ref[1]
    dis = dis_ref[0][:, None]
    h = jnp.maximum(dis * z + b1_ref[...], 0.0)
    y2_ref[...] = jnp.dot(h, w2_ref[...], preferred_element_type=jnp.float32) * dis


def _out_body(zp_ref, dis_ref, b2_ref, o_ref):
    z = (zp_ref[0] + zp_ref[1])[:, : _C]
    dis = dis_ref[0][:, None]
    o = dis * z + b2_ref[...]
    m = jnp.max(o, axis=1, keepdims=True)
    e = jnp.exp(o - m)
    o_ref[...] = o - m - jnp.log(jnp.sum(e, axis=1, keepdims=True))


_scale_matmul = pl.pallas_call(
    _scale_matmul_body,
    grid=(_GRID,),
    in_specs=[
        pl.BlockSpec((_NW, _BN), lambda i: (0, i)),
        pl.BlockSpec((_BN, _D), lambda i: (i, 0)),
        pl.BlockSpec((_D, _H), lambda i: (0, 0)),
    ],
    out_specs=[
        pl.BlockSpec((_BN, _H), lambda i: (i, 0)),
        pl.BlockSpec((1, _BN), lambda i: (0, i)),
    ],
    out_shape=[
        jax.ShapeDtypeStruct((_NP, _H), jnp.float32),
        jax.ShapeDtypeStruct((1, _NP), jnp.float32),
    ],
)

_mid = pl.pallas_call(
    _mid_body,
    grid=(_GRID,),
    in_specs=[
        pl.BlockSpec((_NC, _BN, _H), lambda i: (0, i, 0)),
        pl.BlockSpec((1, _BN), lambda i: (0, i)),
        pl.BlockSpec((1, _H), lambda i: (0, 0)),
        pl.BlockSpec((_H, _H), lambda i: (0, 0)),
    ],
    out_specs=pl.BlockSpec((_BN, _H), lambda i: (i, 0)),
    out_shape=jax.ShapeDtypeStruct((_NP, _H), jnp.float32),
)

_outk = pl.pallas_call(
    _out_body,
    grid=(_GRID,),
    in_specs=[
        pl.BlockSpec((_NC, _BN, _H), lambda i: (0, i, 0)),
        pl.BlockSpec((1, _BN), lambda i: (0, i)),
        pl.BlockSpec((1, _C), lambda i: (0, 0)),
    ],
    out_specs=pl.BlockSpec((_BN, _C), lambda i: (i, 0)),
    out_shape=jax.ShapeDtypeStruct((_NP, _C), jnp.float32),
)


def kernel(x, edge_index, W1, b1, W2, b2):
    E = edge_index.shape[1]
    etot = E + _N
    grain = _NW * _K * _NBUF
    epad = -(-etot // grain) * grain
    nchunks = epad // (_NW * _K)

    loop = jnp.arange(_N, dtype=jnp.int32)
    padv = _N + (jnp.arange(epad - etot, dtype=jnp.int32) % 16)
    src_p = jnp.concatenate([edge_index[0].astype(jnp.int32), loop, padv])
    dst_p = jnp.concatenate([edge_index[1].astype(jnp.int32), loop, padv])

    x_p = jnp.pad(x, ((0, _NP - _N), (0, 0)))
    z128 = jnp.zeros((_NP, _H), jnp.float32)
    W2p = jnp.pad(W2, ((0, 0), (0, _H - _C)))

    srcm = src_p.reshape(_NW, nchunks, _K)

    degp = _make_deg(nchunks)(dst_p)
    y1, dis8 = _scale_matmul(degp, x_p, W1)
    zp1 = _make_agg(nchunks, _H)(y1, srcm, dst_p, z128)
    y2 = _mid(zp1, dis8, b1.reshape(1, _H), W2p)
    zp2 = _make_agg(nchunks, _H)(y2, srcm, dst_p, z128)
    out = _outk(zp2, dis8, b2.reshape(1, _C))
    return out[:_N]


# speedup vs baseline: 33.8481x; 1.0639x over previous
import functools

import jax
import jax.numpy as jnp
from jax import lax
from jax.experimental import pallas as pl
from jax.experimental.pallas import tpu as pltpu
from jax.experimental.pallas import tpu_sc as plsc

_N = 10000
_D = 128
_H = 128
_C = 64

_NC, _NS = 2, 16
_NW = _NC * _NS
_K = 128
_NP = 10240
_ZR = _NP // _NS
_DEGW = 16
_BN = 1280
_GRID = _NP // _BN

_mesh = plsc.VectorSubcoreMesh(core_axis_name="c", subcore_axis_name="s")


def _deg_body(nchunks, dst_hbm, out_hbm, idx_buf, hist):
    c = lax.axis_index("c")
    s = lax.axis_index("s")
    wid = c * _NS + s
    epw = nchunks * _K

    def zero(i, carry):
        hist[pl.ds(i * 16, 16)] = jnp.zeros((16,), jnp.float32)
        return carry

    lax.fori_loop(0, _NP // 16, zero, 0)
    pltpu.sync_copy(dst_hbm.at[pl.ds(wid * epw, epw)], idx_buf)

    def body(i, carry):
        idx = idx_buf[pl.ds(i * 16, 16)]
        cnt, last = plsc.scan_count(idx)
        plsc.addupdate_scatter(hist, [idx], cnt.astype(jnp.float32), mask=last)
        return carry

    lax.fori_loop(0, epw // 16, body, 0)
    pltpu.sync_copy(hist, out_hbm.at[wid])


_NBUF = 2


def _agg_body(nchunks, y_hbm, src_hbm, dst_hbm, zeros_hbm, out_hbm,
              sidx, didx0, didx1, rows, shared, gsem0, gsem1, dsem0, dsem1):
    c = lax.axis_index("c")
    s = lax.axis_index("s")
    wid = c * _NS + s
    epw = nchunks * _K
    didxs = [didx0, didx1]
    gsems = [gsem0, gsem1]
    dsems = [dsem0, dsem1]
    ngrp = nchunks // _NBUF

    pltpu.sync_copy(zeros_hbm.at[pl.ds(s * _ZR, _ZR)], shared.at[pl.ds(s * _ZR, _ZR)])
    pltpu.sync_copy(src_hbm.at[pl.ds(wid * epw, epw)], sidx)
    plsc.subcore_barrier()

    for b in range(_NBUF):
        pltpu.async_copy(dst_hbm.at[pl.ds(wid * epw + b * _K, _K)], didxs[b], dsems[b])
        pltpu.async_copy(y_hbm.at[sidx.at[pl.ds(b * _K, _K)]], rows.at[b], gsems[b])

    def outer(g, carry):
        for b in range(_NBUF):
            j = g * _NBUF + b
            pltpu.make_async_copy(
                dst_hbm.at[pl.ds(wid * epw + j * _K, _K)], didxs[b], dsems[b]
            ).wait()
            pltpu.make_async_copy(
                y_hbm.at[sidx.at[pl.ds(j * _K, _K)]], rows.at[b], gsems[b]
            ).wait()
            pltpu.sync_copy(rows.at[b], shared.at[didxs[b]], add=True)

            @pl.when(g < ngrp - 1)
            def _():
                pltpu.async_copy(
                    dst_hbm.at[pl.ds(wid * epw + (j + _NBUF) * _K, _K)],
                    didxs[b], dsems[b],
                )
                pltpu.async_copy(
                    y_hbm.at[sidx.at[pl.ds((j + _NBUF) * _K, _K)]],
                    rows.at[b], gsems[b],
                )

        return carry

    lax.fori_loop(0, ngrp, outer, 0)
    plsc.subcore_barrier()
    pltpu.sync_copy(shared.at[pl.ds(s * _ZR, _ZR)], out_hbm.at[c, pl.ds(s * _ZR, _ZR)])


def _make_deg(nchunks):
    return pl.kernel(
        functools.partial(_deg_body, nchunks),
        out_type=jax.ShapeDtypeStruct((_NW, _NP), jnp.float32),
        mesh=_mesh,
        scratch_types=[
            pltpu.VMEM((nchunks * _K,), jnp.int32),
            pltpu.VMEM((_NP,), jnp.float32),
        ],
        compiler_params=pltpu.CompilerParams(needs_layout_passes=False),
    )


def _make_agg(nchunks, width, linear=False):
    return pl.kernel(
        functools.partial(_agg_body, nchunks),
        out_type=jax.ShapeDtypeStruct((_NC, _NP, width), jnp.float32),
        mesh=_mesh,
        scratch_types=[
            pltpu.VMEM((nchunks * _K,), jnp.int32),
            pltpu.VMEM((_K,), jnp.int32),
            pltpu.VMEM((_K,), jnp.int32),
            pltpu.VMEM((_NBUF, _K, width), jnp.float32),
            pltpu.VMEM_SHARED((_NP, width), jnp.float32),
            pltpu.SemaphoreType.DMA,
            pltpu.SemaphoreType.DMA,
            pltpu.SemaphoreType.DMA,
            pltpu.SemaphoreType.DMA,
        ],
        compiler_params=(
            pltpu.CompilerParams(use_tc_tiling_on_sc=False) if linear else None
        ),
    )


def _scale_matmul_body(degp_ref, x_ref, w_ref, y_ref, dis_ref):
    deg = jnp.sum(degp_ref[...], axis=0)
    dis = jnp.where(deg > 0, lax.rsqrt(deg), 0.0)
    dis_ref[...] = dis[None, :]
    xw = jnp.dot(x_ref[...], w_ref[...], preferred_element_type=jnp.float32)
    y_ref[...] = xw * dis[:, None]


def _mid_body(zp_ref, dis_ref, b1_ref, w2_ref, y2_ref):
    z = zp_ref[0] + zp_ref[1]
    dis = dis_ref[0][:, None]
    h = jnp.maximum(dis * z + b1_ref[...], 0.0)
    y2_ref[...] = jnp.dot(h, w2_ref[...], preferred_element_type=jnp.float32) * dis


def _out_body(zp_ref, dis_ref, b2_ref, o_ref):
    z = zp_ref[0] + zp_ref[1]
    dis = dis_ref[0][:, None]
    o = dis * z + b2_ref[...]
    m = jnp.max(o, axis=1, keepdims=True)
    e = jnp.exp(o - m)
    o_ref[...] = o - m - jnp.log(jnp.sum(e, axis=1, keepdims=True))


_scale_matmul = pl.pallas_call(
    _scale_matmul_body,
    grid=(_GRID,),
    in_specs=[
        pl.BlockSpec((_NW, _BN), lambda i: (0, i)),
        pl.BlockSpec((_BN, _D), lambda i: (i, 0)),
        pl.BlockSpec((_D, _H), lambda i: (0, 0)),
    ],
    out_specs=[
        pl.BlockSpec((_BN, _H), lambda i: (i, 0)),
        pl.BlockSpec((1, _BN), lambda i: (0, i)),
    ],
    out_shape=[
        jax.ShapeDtypeStruct((_NP, _H), jnp.float32),
        jax.ShapeDtypeStruct((1, _NP), jnp.float32),
    ],
)

_mid = pl.pallas_call(
    _mid_body,
    grid=(_GRID,),
    in_specs=[
        pl.BlockSpec((_NC, _BN, _H), lambda i: (0, i, 0)),
        pl.BlockSpec((1, _BN), lambda i: (0, i)),
        pl.BlockSpec((1, _H), lambda i: (0, 0)),
        pl.BlockSpec((_H, _C), lambda i: (0, 0)),
    ],
    out_specs=pl.BlockSpec((_BN, _C), lambda i: (i, 0)),
    out_shape=jax.ShapeDtypeStruct((_NP, _C), jnp.float32),
)

_outk = pl.pallas_call(
    _out_body,
    grid=(_GRID,),
    in_specs=[
        pl.BlockSpec((_NC, _BN, _C), lambda i: (0, i, 0)),
        pl.BlockSpec((1, _BN), lambda i: (0, i)),
        pl.BlockSpec((1, _C), lambda i: (0, 0)),
    ],
    out_specs=pl.BlockSpec((_BN, _C), lambda i: (i, 0)),
    out_shape=jax.ShapeDtypeStruct((_NP, _C), jnp.float32),
)


def kernel(x, edge_index, W1, b1, W2, b2):
    E = edge_index.shape[1]
    etot = E + _N
    grain = _NW * _K * _NBUF
    epad = -(-etot // grain) * grain
    nchunks = epad // (_NW * _K)

    loop = jnp.arange(_N, dtype=jnp.int32)
    padv = _N + (jnp.arange(epad - etot, dtype=jnp.int32) % 16)
    src_p = jnp.concatenate([edge_index[0].astype(jnp.int32), loop, padv])
    dst_p = jnp.concatenate([edge_index[1].astype(jnp.int32), loop, padv])

    x_p = jnp.pad(x, ((0, _NP - _N), (0, 0)))
    z128 = jnp.zeros((_NP, _H), jnp.float32)
    z64 = jnp.zeros((_NP, _C), jnp.float32)

    degp = _make_deg(nchunks)(dst_p)
    y1, dis8 = _scale_matmul(degp, x_p, W1)
    zp1 = _make_agg(nchunks, _H)(y1, src_p, dst_p, z128)
    y2 = _mid(zp1, dis8, b1.reshape(1, _H), W2)
    zp2 = _make_agg(nchunks, _C, linear=True)(y2, src_p, dst_p, z64)
    out = _outk(zp2, dis8, b2.reshape(1, _C))
    return out[:_N]


# speedup vs baseline: 35.1632x; 1.0389x over previous
import functools

import jax
import jax.numpy as jnp
from jax import lax
from jax.experimental import pallas as pl
from jax.experimental.pallas import tpu as pltpu
from jax.experimental.pallas import tpu_sc as plsc

_N = 10000
_D = 128
_H = 128
_C = 64

_NC, _NS = 2, 16
_NW = _NC * _NS
_K = 128
_NP = 10240
_ZR = _NP // _NS
_BN = 1280
_GRID = _NP // _BN

_mesh = plsc.VectorSubcoreMesh(core_axis_name="c", subcore_axis_name="s")


def _deg_body(nchunks, dst_hbm, out_hbm, idx_buf, hist):
    c = lax.axis_index("c")
    s = lax.axis_index("s")
    wid = c * _NS + s
    epw = nchunks * _K

    def zero(i, carry):
        hist[pl.ds(i * 16, 16)] = jnp.zeros((16,), jnp.float32)
        return carry

    lax.fori_loop(0, _NP // 16, zero, 0)
    pltpu.sync_copy(dst_hbm.at[pl.ds(wid * epw, epw)], idx_buf)

    def body(i, carry):
        idx = idx_buf[pl.ds(i * 16, 16)]
        cnt, last = plsc.scan_count(idx)
        plsc.addupdate_scatter(hist, [idx], cnt.astype(jnp.float32), mask=last)
        return carry

    lax.fori_loop(0, epw // 16, body, 0)
    pltpu.sync_copy(hist, out_hbm.at[wid])


_NBUF = 2


def _agg_body(nchunks, width, y_hbm, src_hbm, dst_hbm, out_hbm,
              sidx, didx0, didx1, rows, shared, gsem0, gsem1, dsem0, dsem1):
    c = lax.axis_index("c")
    s = lax.axis_index("s")
    wid = c * _NS + s
    epw = nchunks * _K
    didxs = [didx0, didx1]
    gsems = [gsem0, gsem1]
    dsems = [dsem0, dsem1]
    ngrp = nchunks // _NBUF

    def zrow(r, carry):
        for q in range(width // 16):
            rows[0, r, pl.ds(q * 16, 16)] = jnp.zeros((16,), jnp.float32)
        return carry

    lax.fori_loop(0, _K, zrow, 0)
    for q in range(_ZR // _K):
        pltpu.sync_copy(rows.at[0], shared.at[pl.ds(s * _ZR + q * _K, _K)])
    pltpu.sync_copy(src_hbm.at[pl.ds(wid * epw, epw)], sidx)
    plsc.subcore_barrier()

    for b in range(_NBUF):
        pltpu.async_copy(dst_hbm.at[pl.ds(wid * epw + b * _K, _K)], didxs[b], dsems[b])
        pltpu.async_copy(y_hbm.at[sidx.at[pl.ds(b * _K, _K)]], rows.at[b], gsems[b])

    def outer(g, carry):
        for b in range(_NBUF):
            j = g * _NBUF + b
            pltpu.make_async_copy(
                dst_hbm.at[pl.ds(wid * epw + j * _K, _K)], didxs[b], dsems[b]
            ).wait()
            pltpu.make_async_copy(
                y_hbm.at[sidx.at[pl.ds(j * _K, _K)]], rows.at[b], gsems[b]
            ).wait()
            pltpu.sync_copy(rows.at[b], shared.at[didxs[b]], add=True)

            @pl.when(g < ngrp - 1)
            def _():
                pltpu.async_copy(
                    dst_hbm.at[pl.ds(wid * epw + (j + _NBUF) * _K, _K)],
                    didxs[b], dsems[b],
                )
                pltpu.async_copy(
                    y_hbm.at[sidx.at[pl.ds((j + _NBUF) * _K, _K)]],
                    rows.at[b], gsems[b],
                )

        return carry

    lax.fori_loop(0, ngrp, outer, 0)
    plsc.subcore_barrier()
    pltpu.sync_copy(shared.at[pl.ds(s * _ZR, _ZR)], out_hbm.at[c, pl.ds(s * _ZR, _ZR)])


def _make_deg(nchunks):
    return pl.kernel(
        functools.partial(_deg_body, nchunks),
        out_type=jax.ShapeDtypeStruct((_NW, _NP), jnp.float32),
        mesh=_mesh,
        scratch_types=[
            pltpu.VMEM((nchunks * _K,), jnp.int32),
            pltpu.VMEM((_NP,), jnp.float32),
        ],
        compiler_params=pltpu.CompilerParams(needs_layout_passes=False),
    )


def _make_agg(nchunks, width, linear=False):
    return pl.kernel(
        functools.partial(_agg_body, nchunks, width),
        out_type=jax.ShapeDtypeStruct((_NC, _NP, width), jnp.float32),
        mesh=_mesh,
        scratch_types=[
            pltpu.VMEM((nchunks * _K,), jnp.int32),
            pltpu.VMEM((_K,), jnp.int32),
            pltpu.VMEM((_K,), jnp.int32),
            pltpu.VMEM((_NBUF, _K, width), jnp.float32),
            pltpu.VMEM_SHARED((_NP, width), jnp.float32),
            pltpu.SemaphoreType.DMA,
            pltpu.SemaphoreType.DMA,
            pltpu.SemaphoreType.DMA,
            pltpu.SemaphoreType.DMA,
        ],
        compiler_params=(
            pltpu.CompilerParams(use_tc_tiling_on_sc=False) if linear else None
        ),
    )


def _scale_matmul_body(degp_ref, x_ref, w_ref, y_ref, dis_ref):
    deg = jnp.sum(degp_ref[...], axis=0)
    dis = jnp.where(deg > 0, lax.rsqrt(deg), 0.0)
    dis_ref[...] = dis[None, :]
    xw = jnp.dot(x_ref[...], w_ref[...], preferred_element_type=jnp.float32)
    y_ref[...] = xw * dis[:, None]


def _mid_body(zp_ref, dis_ref, b1_ref, w2_ref, y2_ref):
    z = zp_ref[0] + zp_ref[1]
    dis = dis_ref[0][:, None]
    h = jnp.maximum(dis * z + b1_ref[...], 0.0)
    y2_ref[...] = jnp.dot(h, w2_ref[...], preferred_element_type=jnp.float32) * dis


def _out_body(zp_ref, dis_ref, b2_ref, o_ref):
    z = zp_ref[0] + zp_ref[1]
    dis = dis_ref[0][:, None]
    o = dis * z + b2_ref[...]
    m = jnp.max(o, axis=1, keepdims=True)
    e = jnp.exp(o - m)
    o_ref[...] = o - m - jnp.log(jnp.sum(e, axis=1, keepdims=True))


_scale_matmul = pl.pallas_call(
    _scale_matmul_body,
    grid=(_GRID,),
    in_specs=[
        pl.BlockSpec((_NW, _BN), lambda i: (0, i)),
        pl.BlockSpec((_BN, _D), lambda i: (i, 0)),
        pl.BlockSpec((_D, _H), lambda i: (0, 0)),
    ],
    out_specs=[
        pl.BlockSpec((_BN, _H), lambda i: (i, 0)),
        pl.BlockSpec((1, _BN), lambda i: (0, i)),
    ],
    out_shape=[
        jax.ShapeDtypeStruct((_NP, _H), jnp.float32),
        jax.ShapeDtypeStruct((1, _NP), jnp.float32),
    ],
)

_mid = pl.pallas_call(
    _mid_body,
    grid=(_GRID,),
    in_specs=[
        pl.BlockSpec((_NC, _BN, _H), lambda i: (0, i, 0)),
        pl.BlockSpec((1, _BN), lambda i: (0, i)),
        pl.BlockSpec((1, _H), lambda i: (0, 0)),
        pl.BlockSpec((_H, _C), lambda i: (0, 0)),
    ],
    out_specs=pl.BlockSpec((_BN, _C), lambda i: (i, 0)),
    out_shape=jax.ShapeDtypeStruct((_NP, _C), jnp.float32),
)

_outk = pl.pallas_call(
    _out_body,
    grid=(_GRID,),
    in_specs=[
        pl.BlockSpec((_NC, _BN, _C), lambda i: (0, i, 0)),
        pl.BlockSpec((1, _BN), lambda i: (0, i)),
        pl.BlockSpec((1, _C), lambda i: (0, 0)),
    ],
    out_specs=pl.BlockSpec((_BN, _C), lambda i: (i, 0)),
    out_shape=jax.ShapeDtypeStruct((_NP, _C), jnp.float32),
)


def kernel(x, edge_index, W1, b1, W2, b2):
    E = edge_index.shape[1]
    etot = E + _N
    grain = _NW * _K * _NBUF
    epad = -(-etot // grain) * grain
    nchunks = epad // (_NW * _K)

    loop = jnp.arange(_N, dtype=jnp.int32)
    padi = jnp.arange(epad - etot, dtype=jnp.int32) % 16
    src_p = jnp.concatenate([edge_index[0].astype(jnp.int32), loop, padi])
    dst_p = jnp.concatenate([edge_index[1].astype(jnp.int32), loop, _N + padi])

    x_p = jnp.pad(x, ((0, _NP - _N), (0, 0)))

    degp = _make_deg(nchunks)(dst_p)
    y1, dis8 = _scale_matmul(degp, x_p, W1)
    zp1 = _make_agg(nchunks, _H)(y1, src_p, dst_p)
    y2 = _mid(zp1, dis8, b1.reshape(1, _H), W2)
    zp2 = _make_agg(nchunks, _C, linear=True)(y2, src_p, dst_p)
    return _outk(zp2, dis8, b2.reshape(1, _C))[:_N]
